# baseline (speedup 1.0000x reference)
import math

import jax
import jax.numpy as jnp
from jax import lax
from jax.experimental import pallas as pl
from jax.experimental.pallas import tpu as pltpu
from jax.experimental.pallas import tpu_sc as plsc

_SQRT_2_OVER_PI = math.sqrt(2.0 / math.pi)

_N = 8192
_D = 768
_K = 512
_NC = 2
_NS = 16
_NW = _NC * _NS
_TOK_PER_TILE = _N // _NW
_LANES = 16


def _phase_a(x_ref, p_ref, assign_ref):
    x = x_ref[...]
    p0 = p_ref[...]
    pn = p0 / jnp.maximum(
        jnp.sqrt(jnp.sum(p0 * p0, axis=1, keepdims=True)), 1e-12)
    rn = jnp.sqrt(jnp.sum(x * x, axis=1, keepdims=True))
    xn = x / jnp.maximum(rn, 1e-8)
    sims = jnp.clip(
        lax.dot_general(xn, pn, (((1,), (1,)), ((), ())),
                        preferred_element_type=jnp.float32),
        -1.0, 1.0)
    m = jnp.max(sims, axis=1, keepdims=True)
    k_iota = lax.broadcasted_iota(jnp.int32, sims.shape, 1)
    idx = jnp.min(jnp.where(sims >= m, k_iota, sims.shape[1]), axis=1)
    assign_ref[...] = idx.reshape(assign_ref.shape)


def _counts_body(a_hbm, counts_hbm, idx_v, hist_v, out_v):
    c = lax.axis_index("c")
    s = lax.axis_index("s")
    w = c * _NS + s
    pltpu.sync_copy(a_hbm.at[pl.ds(w * _TOK_PER_TILE, _TOK_PER_TILE)], idx_v)

    def zero(i, _):
        def zr(r, _):
            hist_v[r, pl.ds(i * _LANES, _LANES)] = jnp.zeros(
                (_LANES,), jnp.float32)
            return 0
        lax.fori_loop(0, _LANES, zr, 0)
        return 0
    lax.fori_loop(0, _K // _LANES, zero, 0)

    lane_iota = lax.iota(jnp.int32, _LANES)
    ones16 = jnp.ones((_LANES,), jnp.float32)

    def accum(i, _):
        iv = idx_v[pl.ds(i * _LANES, _LANES)]
        plsc.addupdate_scatter(hist_v, [lane_iota, iv], ones16)
        return 0
    lax.fori_loop(0, _TOK_PER_TILE // _LANES, accum, 0)

    def reduce_cols(j, _):
        def rr(r, acc):
            return acc + hist_v[r, pl.ds(j * _LANES, _LANES)]
        out_v[pl.ds(j * _LANES, _LANES)] = lax.fori_loop(
            0, _LANES, rr, jnp.zeros((_LANES,), jnp.float32))
        return 0
    lax.fori_loop(0, _K // _LANES, reduce_cols, 0)
    pltpu.sync_copy(out_v, counts_hbm.at[w])


def _sums_kernel(x_ref, a_ref, sums_ref):
    i = pl.program_id(0)
    x = x_ref[...]
    a = a_ref[...].reshape(x.shape[0], 1)
    k_iota = lax.broadcasted_iota(jnp.int32, (x.shape[0], _K), 1)
    e = (k_iota == a).astype(jnp.float32)
    part = lax.dot_general(e, x, (((0,), (0,)), ((), ())),
                           preferred_element_type=jnp.float32)

    @pl.when(i == 0)
    def _():
        sums_ref[...] = jnp.zeros_like(sums_ref)

    sums_ref[...] += part


def _phase_b(sums_ref, counts_ref, p_ref, out_ref):
    momentum = 0.999
    p0 = p_ref[...]
    sums = sums_ref[...]
    counts = lax.dot_general(
        counts_ref[...], jnp.ones((_NW, 1), jnp.float32),
        (((0,), (0,)), ((), ())),
        preferred_element_type=jnp.float32)
    centroids = jnp.where(counts > 0.0, sums / jnp.maximum(counts, 1.0), p0)
    new_p = centroids / jnp.maximum(
        jnp.sqrt(jnp.sum(centroids * centroids, axis=1, keepdims=True)), 1e-12)
    p_upd = momentum * p0 + (1.0 - momentum) * new_p
    out_ref[...] = p_upd / jnp.maximum(
        jnp.sqrt(jnp.sum(p_upd * p_upd, axis=1, keepdims=True)), 1e-8)


def _phase_c(lt_ref, lb_ref, x_ref, pn2_ref, out_ref):
    x = x_ref[...]
    pn2 = pn2_ref[...]
    rn = jnp.sqrt(jnp.sum(x * x, axis=1, keepdims=True))
    xn = x / jnp.maximum(rn, 1e-8)
    sims2 = jnp.clip(
        lax.dot_general(xn, pn2, (((1,), (1,)), ((), ())),
                        preferred_element_type=jnp.float32),
        -1.0, 1.0)
    mx = jnp.max(sims2, axis=1, keepdims=True)
    dists = jnp.clip(1.0 - mx, 0.0, 2.0)
    tau = jnp.exp(lt_ref[0])
    alpha = jax.nn.sigmoid(lb_ref[0])
    novelty = 1.0 - jnp.exp(-tau * dists)
    scale = jnp.clip(1.0 - alpha + alpha * novelty, 0.1, 10.0)
    y = x * scale
    out_ref[...] = 0.5 * y * (
        1.0 + jnp.tanh(_SQRT_2_OVER_PI * (y + 0.044715 * y * y * y)))


def kernel(x, P, log_tau, log_blend):
    B, T, D = x.shape
    K = P.shape[0]
    N = B * T
    xf = x.reshape(N, D)
    TT = 512
    n_tiles = N // TT

    assign = pl.pallas_call(
        _phase_a,
        grid=(n_tiles,),
        in_specs=[
            pl.BlockSpec((TT, D), lambda i: (i, 0)),
            pl.BlockSpec((K, D), lambda i: (0, 0)),
        ],
        out_specs=pl.BlockSpec((1, 1, TT), lambda i: (i, 0, 0)),
        out_shape=jax.ShapeDtypeStruct((n_tiles, 1, TT), jnp.int32),
    )(xf, P)

    counts_p = pl.kernel(
        _counts_body,
        out_type=jax.ShapeDtypeStruct((_NW, _K), jnp.float32),
        mesh=plsc.VectorSubcoreMesh(core_axis_name="c", subcore_axis_name="s"),
        compiler_params=pltpu.CompilerParams(needs_layout_passes=False),
        scratch_types=[
            pltpu.VMEM((_TOK_PER_TILE,), jnp.int32),
            pltpu.VMEM((_LANES, _K), jnp.float32),
            pltpu.VMEM((_K,), jnp.float32),
        ],
    )(assign.reshape(N))

    sums = pl.pallas_call(
        _sums_kernel,
        grid=(n_tiles,),
        in_specs=[
            pl.BlockSpec((TT, D), lambda i: (i, 0)),
            pl.BlockSpec((1, 1, TT), lambda i: (i, 0, 0)),
        ],
        out_specs=pl.BlockSpec((K, D), lambda i: (0, 0)),
        out_shape=jax.ShapeDtypeStruct((K, D), jnp.float32),
    )(xf, assign)

    pn2 = pl.pallas_call(
        _phase_b,
        out_shape=jax.ShapeDtypeStruct((K, D), jnp.float32),
    )(sums, counts_p, P)

    lt = jnp.reshape(log_tau, (1,))
    lb = jnp.reshape(log_blend, (1,))
    out = pl.pallas_call(
        _phase_c,
        grid=(n_tiles,),
        in_specs=[
            pl.BlockSpec(memory_space=pltpu.SMEM),
            pl.BlockSpec(memory_space=pltpu.SMEM),
            pl.BlockSpec((TT, D), lambda i: (i, 0)),
            pl.BlockSpec((K, D), lambda i: (0, 0)),
        ],
        out_specs=pl.BlockSpec((TT, D), lambda i: (i, 0)),
        out_shape=jax.ShapeDtypeStruct((N, D), jnp.float32),
    )(lt, lb, xf, pn2)

    return out.reshape(B, T, D)


# speedup vs baseline: 1.0915x; 1.0915x over previous
import math

import jax
import jax.numpy as jnp
from jax import lax
from jax.experimental import pallas as pl
from jax.experimental.pallas import tpu as pltpu
from jax.experimental.pallas import tpu_sc as plsc

_SQRT_2_OVER_PI = math.sqrt(2.0 / math.pi)

_N = 8192
_D = 768
_K = 512
_NC = 2
_NS = 16
_NW = _NC * _NS
_TOK_PER_TILE = _N // _NW
_LANES = 16


def _phase_a(x_ref, p_ref, assign_ref, sums_ref):
    i = pl.program_id(0)
    x = x_ref[...]
    p0 = p_ref[...]
    pn = p0 / jnp.maximum(
        jnp.sqrt(jnp.sum(p0 * p0, axis=1, keepdims=True)), 1e-12)
    rn = jnp.sqrt(jnp.sum(x * x, axis=1, keepdims=True))
    xn = x / jnp.maximum(rn, 1e-8)
    sims = jnp.clip(
        lax.dot_general(xn.astype(jnp.bfloat16), pn.astype(jnp.bfloat16),
                        (((1,), (1,)), ((), ())),
                        preferred_element_type=jnp.float32),
        -1.0, 1.0)
    m = jnp.max(sims, axis=1, keepdims=True)
    k_iota = lax.broadcasted_iota(jnp.int32, sims.shape, 1)
    idx = jnp.min(jnp.where(sims >= m, k_iota, sims.shape[1]), axis=1)
    assign_ref[...] = idx.reshape(assign_ref.shape)
    e = (k_iota == idx[:, None]).astype(jnp.float32)
    part = lax.dot_general(e, x, (((0,), (0,)), ((), ())),
                           preferred_element_type=jnp.float32)

    @pl.when(i == 0)
    def _():
        sums_ref[...] = jnp.zeros_like(sums_ref)

    sums_ref[...] += part


def _counts_body(a_hbm, counts_hbm, idx_v, hist_v, out_v):
    c = lax.axis_index("c")
    s = lax.axis_index("s")
    w = c * _NS + s
    pltpu.sync_copy(a_hbm.at[pl.ds(w * _TOK_PER_TILE, _TOK_PER_TILE)], idx_v)

    def zero(i, _):
        def zr(r, _):
            hist_v[r, pl.ds(i * _LANES, _LANES)] = jnp.zeros(
                (_LANES,), jnp.float32)
            return 0
        lax.fori_loop(0, _LANES, zr, 0)
        return 0
    lax.fori_loop(0, _K // _LANES, zero, 0)

    lane_iota = lax.iota(jnp.int32, _LANES)
    ones16 = jnp.ones((_LANES,), jnp.float32)

    def accum(i, _):
        iv = idx_v[pl.ds(i * _LANES, _LANES)]
        plsc.addupdate_scatter(hist_v, [lane_iota, iv], ones16)
        return 0
    lax.fori_loop(0, _TOK_PER_TILE // _LANES, accum, 0)

    def reduce_cols(j, _):
        def rr(r, acc):
            return acc + hist_v[r, pl.ds(j * _LANES, _LANES)]
        out_v[pl.ds(j * _LANES, _LANES)] = lax.fori_loop(
            0, _LANES, rr, jnp.zeros((_LANES,), jnp.float32))
        return 0
    lax.fori_loop(0, _K // _LANES, reduce_cols, 0)
    pltpu.sync_copy(out_v, counts_hbm.at[w])


def _phase_b(sums_ref, counts_ref, p_ref, out_ref):
    momentum = 0.999
    p0 = p_ref[...]
    sums = sums_ref[...]
    counts = lax.dot_general(
        counts_ref[...], jnp.ones((_NW, 1), jnp.float32),
        (((0,), (0,)), ((), ())),
        preferred_element_type=jnp.float32)
    centroids = jnp.where(counts > 0.0, sums / jnp.maximum(counts, 1.0), p0)
    new_p = centroids / jnp.maximum(
        jnp.sqrt(jnp.sum(centroids * centroids, axis=1, keepdims=True)), 1e-12)
    p_upd = momentum * p0 + (1.0 - momentum) * new_p
    out_ref[...] = p_upd / jnp.maximum(
        jnp.sqrt(jnp.sum(p_upd * p_upd, axis=1, keepdims=True)), 1e-8)


def _phase_c(lt_ref, lb_ref, x_ref, pn2_ref, out_ref):
    x = x_ref[...]
    pn2 = pn2_ref[...]
    rn = jnp.sqrt(jnp.sum(x * x, axis=1, keepdims=True))
    xn = x / jnp.maximum(rn, 1e-8)
    sims2 = jnp.clip(
        lax.dot_general(xn.astype(jnp.bfloat16), pn2.astype(jnp.bfloat16),
                        (((1,), (1,)), ((), ())),
                        preferred_element_type=jnp.float32),
        -1.0, 1.0)
    mx = jnp.max(sims2, axis=1, keepdims=True)
    dists = jnp.clip(1.0 - mx, 0.0, 2.0)
    tau = jnp.exp(lt_ref[0])
    alpha = jax.nn.sigmoid(lb_ref[0])
    novelty = 1.0 - jnp.exp(-tau * dists)
    scale = jnp.clip(1.0 - alpha + alpha * novelty, 0.1, 10.0)
    y = x * scale
    out_ref[...] = 0.5 * y * (
        1.0 + jnp.tanh(_SQRT_2_OVER_PI * (y + 0.044715 * y * y * y)))


def kernel(x, P, log_tau, log_blend):
    B, T, D = x.shape
    K = P.shape[0]
    N = B * T
    xf = x.reshape(N, D)
    TT = 512
    n_tiles = N // TT

    assign, sums = pl.pallas_call(
        _phase_a,
        grid=(n_tiles,),
        in_specs=[
            pl.BlockSpec((TT, D), lambda i: (i, 0)),
            pl.BlockSpec((K, D), lambda i: (0, 0)),
        ],
        out_specs=[
            pl.BlockSpec((1, 1, TT), lambda i: (i, 0, 0)),
            pl.BlockSpec((K, D), lambda i: (0, 0)),
        ],
        out_shape=[
            jax.ShapeDtypeStruct((n_tiles, 1, TT), jnp.int32),
            jax.ShapeDtypeStruct((K, D), jnp.float32),
        ],
    )(xf, P)

    counts_p = pl.kernel(
        _counts_body,
        out_type=jax.ShapeDtypeStruct((_NW, _K), jnp.float32),
        mesh=plsc.VectorSubcoreMesh(core_axis_name="c", subcore_axis_name="s"),
        compiler_params=pltpu.CompilerParams(needs_layout_passes=False),
        scratch_types=[
            pltpu.VMEM((_TOK_PER_TILE,), jnp.int32),
            pltpu.VMEM((_LANES, _K), jnp.float32),
            pltpu.VMEM((_K,), jnp.float32),
        ],
    )(assign.reshape(N))

    pn2 = pl.pallas_call(
        _phase_b,
        out_shape=jax.ShapeDtypeStruct((K, D), jnp.float32),
    )(sums, counts_p, P)

    lt = jnp.reshape(log_tau, (1,))
    lb = jnp.reshape(log_blend, (1,))
    out = pl.pallas_call(
        _phase_c,
        grid=(n_tiles,),
        in_specs=[
            pl.BlockSpec(memory_space=pltpu.SMEM),
            pl.BlockSpec(memory_space=pltpu.SMEM),
            pl.BlockSpec((TT, D), lambda i: (i, 0)),
            pl.BlockSpec((K, D), lambda i: (0, 0)),
        ],
        out_specs=pl.BlockSpec((TT, D), lambda i: (i, 0)),
        out_shape=jax.ShapeDtypeStruct((N, D), jnp.float32),
    )(lt, lb, xf, pn2)

    return out.reshape(B, T, D)


# speedup vs baseline: 1.2531x; 1.1481x over previous
import math

import jax
import jax.numpy as jnp
from jax import lax
from jax.experimental import pallas as pl
from jax.experimental.pallas import tpu as pltpu
from jax.experimental.pallas import tpu_sc as plsc

_SQRT_2_OVER_PI = math.sqrt(2.0 / math.pi)

_N = 8192
_D = 768
_K = 512
_NC = 2
_NS = 16
_NW = _NC * _NS
_LANES = 16
_HALF = _N // 2
_SC_TOK = _HALF // _NW


def _phase_a(x_ref, p_ref, assign_ref, sums_ref, counts_ref=None):
    i = pl.program_id(0)
    x = x_ref[...]
    p0 = p_ref[...]
    pn = p0 / jnp.maximum(
        jnp.sqrt(jnp.sum(p0 * p0, axis=1, keepdims=True)), 1e-12)
    rn = jnp.sqrt(jnp.sum(x * x, axis=1, keepdims=True))
    xn = x / jnp.maximum(rn, 1e-8)
    sims = jnp.clip(
        lax.dot_general(xn.astype(jnp.bfloat16), pn.astype(jnp.bfloat16),
                        (((1,), (1,)), ((), ())),
                        preferred_element_type=jnp.float32),
        -1.0, 1.0)
    m = jnp.max(sims, axis=1, keepdims=True)
    k_iota = lax.broadcasted_iota(jnp.int32, sims.shape, 1)
    idx = jnp.min(jnp.where(sims >= m, k_iota, sims.shape[1]), axis=1)
    assign_ref[...] = idx.reshape(assign_ref.shape)
    e = (k_iota == idx[:, None]).astype(jnp.bfloat16)
    part = lax.dot_general(e, x.astype(jnp.bfloat16), (((0,), (0,)), ((), ())),
                           preferred_element_type=jnp.float32)
    if counts_ref is not None:
        cpart = lax.dot_general(e, jnp.ones((x.shape[0], 1), jnp.bfloat16),
                                (((0,), (0,)), ((), ())),
                                preferred_element_type=jnp.float32)

    @pl.when(i == 0)
    def _():
        sums_ref[...] = jnp.zeros_like(sums_ref)
        if counts_ref is not None:
            counts_ref[...] = jnp.zeros_like(counts_ref)

    sums_ref[...] += part
    if counts_ref is not None:
        counts_ref[...] += cpart


def _counts_body(a_hbm, counts_hbm, idx_v, hist_v, out_v):
    c = lax.axis_index("c")
    s = lax.axis_index("s")
    w = c * _NS + s
    pltpu.sync_copy(a_hbm.at[pl.ds(w * _SC_TOK, _SC_TOK)], idx_v)

    def zero(i, _):
        def zr(r, _):
            hist_v[r, pl.ds(i * _LANES, _LANES)] = jnp.zeros(
                (_LANES,), jnp.float32)
            return 0
        lax.fori_loop(0, _LANES, zr, 0)
        return 0
    lax.fori_loop(0, _K // _LANES, zero, 0)

    lane_iota = lax.iota(jnp.int32, _LANES)
    ones16 = jnp.ones((_LANES,), jnp.float32)

    def accum(i, _):
        iv = idx_v[pl.ds(i * _LANES, _LANES)]
        plsc.addupdate_scatter(hist_v, [lane_iota, iv], ones16)
        return 0
    lax.fori_loop(0, _SC_TOK // _LANES, accum, 0)

    def reduce_cols(j, _):
        def rr(r, acc):
            return acc + hist_v[r, pl.ds(j * _LANES, _LANES)]
        out_v[pl.ds(j * _LANES, _LANES)] = lax.fori_loop(
            0, _LANES, rr, jnp.zeros((_LANES,), jnp.float32))
        return 0
    lax.fori_loop(0, _K // _LANES, reduce_cols, 0)
    pltpu.sync_copy(out_v, counts_hbm.at[w])


def _phase_b(sums1_ref, sums2_ref, csc_ref, ctc_ref, p_ref, out_ref):
    momentum = 0.999
    p0 = p_ref[...]
    sums = sums1_ref[...] + sums2_ref[...]
    counts = ctc_ref[...] + lax.dot_general(
        csc_ref[...], jnp.ones((_NW, 1), jnp.float32),
        (((0,), (0,)), ((), ())),
        preferred_element_type=jnp.float32)
    centroids = jnp.where(counts > 0.0, sums / jnp.maximum(counts, 1.0), p0)
    new_p = centroids / jnp.maximum(
        jnp.sqrt(jnp.sum(centroids * centroids, axis=1, keepdims=True)), 1e-12)
    p_upd = momentum * p0 + (1.0 - momentum) * new_p
    out_ref[...] = p_upd / jnp.maximum(
        jnp.sqrt(jnp.sum(p_upd * p_upd, axis=1, keepdims=True)), 1e-8)


def _phase_c(lt_ref, lb_ref, x_ref, pn2_ref, out_ref):
    x = x_ref[...]
    pn2 = pn2_ref[...]
    rn = jnp.sqrt(jnp.sum(x * x, axis=1, keepdims=True))
    xn = x / jnp.maximum(rn, 1e-8)
    sims2 = jnp.clip(
        lax.dot_general(xn.astype(jnp.bfloat16), pn2.astype(jnp.bfloat16),
                        (((1,), (1,)), ((), ())),
                        preferred_element_type=jnp.float32),
        -1.0, 1.0)
    mx = jnp.max(sims2, axis=1, keepdims=True)
    dists = jnp.clip(1.0 - mx, 0.0, 2.0)
    tau = jnp.exp(lt_ref[0])
    alpha = jax.nn.sigmoid(lb_ref[0])
    novelty = 1.0 - jnp.exp(-tau * dists)
    scale = jnp.clip(1.0 - alpha + alpha * novelty, 0.1, 10.0)
    y = x * scale
    out_ref[...] = 0.5 * y * (
        1.0 + jnp.tanh(_SQRT_2_OVER_PI * (y + 0.044715 * y * y * y)))


def kernel(x, P, log_tau, log_blend):
    B, T, D = x.shape
    K = P.shape[0]
    N = B * T
    xf = x.reshape(N, D)
    TT = 1024
    half_tiles = _HALF // TT

    assign1, sums1 = pl.pallas_call(
        _phase_a,
        grid=(half_tiles,),
        in_specs=[
            pl.BlockSpec((TT, D), lambda i: (i, 0)),
            pl.BlockSpec((K, D), lambda i: (0, 0)),
        ],
        out_specs=[
            pl.BlockSpec((1, 1, TT), lambda i: (i, 0, 0)),
            pl.BlockSpec((K, D), lambda i: (0, 0)),
        ],
        out_shape=[
            jax.ShapeDtypeStruct((half_tiles, 1, TT), jnp.int32),
            jax.ShapeDtypeStruct((K, D), jnp.float32),
        ],
    )(xf, P)

    counts_sc = pl.kernel(
        _counts_body,
        out_type=jax.ShapeDtypeStruct((_NW, _K), jnp.float32),
        mesh=plsc.VectorSubcoreMesh(core_axis_name="c", subcore_axis_name="s"),
        compiler_params=pltpu.CompilerParams(needs_layout_passes=False),
        scratch_types=[
            pltpu.VMEM((_SC_TOK,), jnp.int32),
            pltpu.VMEM((_LANES, _K), jnp.float32),
            pltpu.VMEM((_K,), jnp.float32),
        ],
    )(assign1.reshape(_HALF))

    assign2, sums2, counts_tc = pl.pallas_call(
        _phase_a,
        grid=(half_tiles,),
        in_specs=[
            pl.BlockSpec((TT, D), lambda i: (i + _HALF // 1024, 0)),
            pl.BlockSpec((K, D), lambda i: (0, 0)),
        ],
        out_specs=[
            pl.BlockSpec((1, 1, TT), lambda i: (i, 0, 0)),
            pl.BlockSpec((K, D), lambda i: (0, 0)),
            pl.BlockSpec((K, 1), lambda i: (0, 0)),
        ],
        out_shape=[
            jax.ShapeDtypeStruct((half_tiles, 1, TT), jnp.int32),
            jax.ShapeDtypeStruct((K, D), jnp.float32),
            jax.ShapeDtypeStruct((K, 1), jnp.float32),
        ],
    )(xf, P)
    del assign2

    pn2 = pl.pallas_call(
        _phase_b,
        out_shape=jax.ShapeDtypeStruct((K, D), jnp.float32),
    )(sums1, sums2, counts_sc, counts_tc, P)

    lt = jnp.reshape(log_tau, (1,))
    lb = jnp.reshape(log_blend, (1,))
    out = pl.pallas_call(
        _phase_c,
        grid=(N // TT,),
        in_specs=[
            pl.BlockSpec(memory_space=pltpu.SMEM),
            pl.BlockSpec(memory_space=pltpu.SMEM),
            pl.BlockSpec((TT, D), lambda i: (i, 0)),
            pl.BlockSpec((K, D), lambda i: (0, 0)),
        ],
        out_specs=pl.BlockSpec((TT, D), lambda i: (i, 0)),
        out_shape=jax.ShapeDtypeStruct((N, D), jnp.float32),
    )(lt, lb, xf, pn2)

    return out.reshape(B, T, D)


# speedup vs baseline: 1.2998x; 1.0373x over previous
import math

import jax
import jax.numpy as jnp
from jax import lax
from jax.experimental import pallas as pl
from jax.experimental.pallas import tpu as pltpu
from jax.experimental.pallas import tpu_sc as plsc

_SQRT_2_OVER_PI = math.sqrt(2.0 / math.pi)

_N = 8192
_D = 768
_K = 512
_NC = 2
_NS = 16
_NW = _NC * _NS
_LANES = 16
_HALF = _N // 2
_SC_TOK = _HALF // _NW
_TT = 1024
_HALF_TILES = _HALF // _TT


def _phase_a(x_ref, p_ref, assign_ref, sums_ref, counts_ref):
    i = pl.program_id(0)
    x = x_ref[...]
    p0 = p_ref[...]
    pn = p0 / jnp.maximum(
        jnp.sqrt(jnp.sum(p0 * p0, axis=1, keepdims=True)), 1e-12)
    rn = jnp.sqrt(jnp.sum(x * x, axis=1, keepdims=True))
    xn = x / jnp.maximum(rn, 1e-8)
    sims = jnp.clip(
        lax.dot_general(xn.astype(jnp.bfloat16), pn.astype(jnp.bfloat16),
                        (((1,), (1,)), ((), ())),
                        preferred_element_type=jnp.float32),
        -1.0, 1.0)
    m = jnp.max(sims, axis=1, keepdims=True)
    k_iota = lax.broadcasted_iota(jnp.int32, sims.shape, 1)
    idx = jnp.min(jnp.where(sims >= m, k_iota, sims.shape[1]), axis=1)
    assign_ref[...] = idx.reshape(assign_ref.shape)
    e = (k_iota == idx[:, None]).astype(jnp.bfloat16)
    part = lax.dot_general(e, x.astype(jnp.bfloat16), (((0,), (0,)), ((), ())),
                           preferred_element_type=jnp.float32)
    cpart = lax.dot_general(e, jnp.ones((x.shape[0], 1), jnp.bfloat16),
                            (((0,), (0,)), ((), ())),
                            preferred_element_type=jnp.float32)

    @pl.when(i == 0)
    def _():
        sums_ref[...] = jnp.zeros_like(sums_ref)
        counts_ref[...] = jnp.zeros_like(counts_ref)

    sums_ref[...] += part

    @pl.when(i >= _HALF_TILES)
    def _():
        counts_ref[...] += cpart


def _counts_body(a_hbm, counts_hbm, idx_v, hist_v, out_v):
    c = lax.axis_index("c")
    s = lax.axis_index("s")
    w = c * _NS + s
    pltpu.sync_copy(a_hbm.at[pl.ds(w * _SC_TOK, _SC_TOK)], idx_v)

    def zero(i, _):
        def zr(r, _):
            hist_v[r, pl.ds(i * _LANES, _LANES)] = jnp.zeros(
                (_LANES,), jnp.float32)
            return 0
        lax.fori_loop(0, _LANES, zr, 0)
        return 0
    lax.fori_loop(0, _K // _LANES, zero, 0)

    lane_iota = lax.iota(jnp.int32, _LANES)
    ones16 = jnp.ones((_LANES,), jnp.float32)

    def accum(i, _):
        iv = idx_v[pl.ds(i * _LANES, _LANES)]
        plsc.addupdate_scatter(hist_v, [lane_iota, iv], ones16)
        return 0
    lax.fori_loop(0, _SC_TOK // _LANES, accum, 0)

    def reduce_cols(j, _):
        def rr(r, acc):
            return acc + hist_v[r, pl.ds(j * _LANES, _LANES)]
        out_v[pl.ds(j * _LANES, _LANES)] = lax.fori_loop(
            0, _LANES, rr, jnp.zeros((_LANES,), jnp.float32))
        return 0
    lax.fori_loop(0, _K // _LANES, reduce_cols, 0)
    pltpu.sync_copy(out_v, counts_hbm.at[w])


def _phase_c(lt_ref, lb_ref, sums_ref, csc_ref, ctc_ref, p_ref, x_ref,
             out_ref, pn2_scr):
    i = pl.program_id(0)

    @pl.when(i == 0)
    def _():
        momentum = 0.999
        p0 = p_ref[...]
        sums = sums_ref[...]
        counts = ctc_ref[...] + lax.dot_general(
            csc_ref[...], jnp.ones((_NW, 1), jnp.float32),
            (((0,), (0,)), ((), ())),
            preferred_element_type=jnp.float32)
        centroids = jnp.where(counts > 0.0,
                              sums / jnp.maximum(counts, 1.0), p0)
        new_p = centroids / jnp.maximum(
            jnp.sqrt(jnp.sum(centroids * centroids, axis=1, keepdims=True)),
            1e-12)
        p_upd = momentum * p0 + (1.0 - momentum) * new_p
        pn2_scr[...] = p_upd / jnp.maximum(
            jnp.sqrt(jnp.sum(p_upd * p_upd, axis=1, keepdims=True)), 1e-8)

    x = x_ref[...]
    pn2 = pn2_scr[...]
    rn = jnp.sqrt(jnp.sum(x * x, axis=1, keepdims=True))
    xn = x / jnp.maximum(rn, 1e-8)
    sims2 = jnp.clip(
        lax.dot_general(xn.astype(jnp.bfloat16), pn2.astype(jnp.bfloat16),
                        (((1,), (1,)), ((), ())),
                        preferred_element_type=jnp.float32),
        -1.0, 1.0)
    mx = jnp.max(sims2, axis=1, keepdims=True)
    dists = jnp.clip(1.0 - mx, 0.0, 2.0)
    tau = jnp.exp(lt_ref[0])
    alpha = jax.nn.sigmoid(lb_ref[0])
    novelty = 1.0 - jnp.exp(-tau * dists)
    scale = jnp.clip(1.0 - alpha + alpha * novelty, 0.1, 10.0)
    y = x * scale
    out_ref[...] = 0.5 * y * (
        1.0 + jnp.tanh(_SQRT_2_OVER_PI * (y + 0.044715 * y * y * y)))


def kernel(x, P, log_tau, log_blend):
    B, T, D = x.shape
    K = P.shape[0]
    N = B * T
    xf = x.reshape(N, D)
    n_tiles = N // _TT

    assign, sums, counts_tc = pl.pallas_call(
        _phase_a,
        grid=(n_tiles,),
        in_specs=[
            pl.BlockSpec((_TT, D), lambda i: (i, 0)),
            pl.BlockSpec((K, D), lambda i: (0, 0)),
        ],
        out_specs=[
            pl.BlockSpec((1, 1, _TT), lambda i: (i, 0, 0)),
            pl.BlockSpec((K, D), lambda i: (0, 0)),
            pl.BlockSpec((K, 1), lambda i: (0, 0)),
        ],
        out_shape=[
            jax.ShapeDtypeStruct((n_tiles, 1, _TT), jnp.int32),
            jax.ShapeDtypeStruct((K, D), jnp.float32),
            jax.ShapeDtypeStruct((K, 1), jnp.float32),
        ],
    )(xf, P)

    counts_sc = pl.kernel(
        _counts_body,
        out_type=jax.ShapeDtypeStruct((_NW, _K), jnp.float32),
        mesh=plsc.VectorSubcoreMesh(core_axis_name="c", subcore_axis_name="s"),
        compiler_params=pltpu.CompilerParams(needs_layout_passes=False),
        scratch_types=[
            pltpu.VMEM((_SC_TOK,), jnp.int32),
            pltpu.VMEM((_LANES, _K), jnp.float32),
            pltpu.VMEM((_K,), jnp.float32),
        ],
    )(assign.reshape(N)[:_HALF])

    lt = jnp.reshape(log_tau, (1,))
    lb = jnp.reshape(log_blend, (1,))
    out = pl.pallas_call(
        _phase_c,
        grid=(n_tiles,),
        in_specs=[
            pl.BlockSpec(memory_space=pltpu.SMEM),
            pl.BlockSpec(memory_space=pltpu.SMEM),
            pl.BlockSpec((K, D), lambda i: (0, 0)),
            pl.BlockSpec((_NW, K), lambda i: (0, 0)),
            pl.BlockSpec((K, 1), lambda i: (0, 0)),
            pl.BlockSpec((K, D), lambda i: (0, 0)),
            pl.BlockSpec((_TT, D), lambda i: (i, 0)),
        ],
        out_specs=pl.BlockSpec((_TT, D), lambda i: (i, 0)),
        out_shape=jax.ShapeDtypeStruct((N, D), jnp.float32),
        scratch_shapes=[pltpu.VMEM((K, D), jnp.float32)],
    )(lt, lb, sums, counts_sc, counts_tc, P, xf)

    return out.reshape(B, T, D)


# speedup vs baseline: 1.3167x; 1.0130x over previous
import math

import jax
import jax.numpy as jnp
from jax import lax
from jax.experimental import pallas as pl
from jax.experimental.pallas import tpu as pltpu
from jax.experimental.pallas import tpu_sc as plsc

_SQRT_2_OVER_PI = math.sqrt(2.0 / math.pi)

_N = 8192
_D = 768
_K = 512
_NC = 2
_NS = 16
_NW = _NC * _NS
_LANES = 16
_HALF = _N // 2
_SC_TOK = _HALF // _NW
_TT = 1024
_HALF_TILES = _HALF // _TT


def _phase_a(x_ref, p_ref, assign_ref, sums_ref, counts_ref):
    i = pl.program_id(0)
    x = x_ref[...]
    p0 = p_ref[...]
    pn = p0 / jnp.maximum(
        jnp.sqrt(jnp.sum(p0 * p0, axis=1, keepdims=True)), 1e-12)
    rn = jnp.sqrt(jnp.sum(x * x, axis=1, keepdims=True))
    xn = x / jnp.maximum(rn, 1e-8)
    sims = lax.dot_general(xn.astype(jnp.bfloat16), pn.astype(jnp.bfloat16),
                           (((1,), (1,)), ((), ())),
                           preferred_element_type=jnp.float32)
    m = jnp.max(sims, axis=1, keepdims=True)
    k_iota = lax.broadcasted_iota(jnp.int32, sims.shape, 1)
    idx = jnp.min(jnp.where(sims >= m, k_iota, sims.shape[1]), axis=1)
    assign_ref[...] = idx.reshape(assign_ref.shape)
    e = (k_iota == idx[:, None]).astype(jnp.bfloat16)
    part = lax.dot_general(e, x.astype(jnp.bfloat16), (((0,), (0,)), ((), ())),
                           preferred_element_type=jnp.float32)
    cpart = lax.dot_general(e, jnp.ones((x.shape[0], 1), jnp.bfloat16),
                            (((0,), (0,)), ((), ())),
                            preferred_element_type=jnp.float32)

    @pl.when(i == 0)
    def _():
        sums_ref[...] = jnp.zeros_like(sums_ref)
        counts_ref[...] = jnp.zeros_like(counts_ref)

    sums_ref[...] += part

    @pl.when(i >= _HALF_TILES)
    def _():
        counts_ref[...] += cpart


def _counts_body(a_hbm, counts_hbm, idx_v, hist_v, out_v):
    c = lax.axis_index("c")
    s = lax.axis_index("s")
    w = c * _NS + s
    pltpu.sync_copy(a_hbm.at[pl.ds(w * _SC_TOK, _SC_TOK)], idx_v)

    def zero(i, _):
        def zr(r, _):
            hist_v[r, pl.ds(i * _LANES, _LANES)] = jnp.zeros(
                (_LANES,), jnp.float32)
            return 0
        lax.fori_loop(0, _LANES, zr, 0)
        return 0
    lax.fori_loop(0, _K // _LANES, zero, 0)

    lane_iota = lax.iota(jnp.int32, _LANES)
    ones16 = jnp.ones((_LANES,), jnp.float32)

    def accum(i, _):
        iv = idx_v[pl.ds(i * _LANES, _LANES)]
        plsc.addupdate_scatter(hist_v, [lane_iota, iv], ones16)
        return 0
    lax.fori_loop(0, _SC_TOK // _LANES, accum, 0)

    def reduce_cols(j, _):
        def rr(r, acc):
            return acc + hist_v[r, pl.ds(j * _LANES, _LANES)]
        out_v[pl.ds(j * _LANES, _LANES)] = lax.fori_loop(
            0, _LANES, rr, jnp.zeros((_LANES,), jnp.float32))
        return 0
    lax.fori_loop(0, _K // _LANES, reduce_cols, 0)
    pltpu.sync_copy(out_v, counts_hbm.at[w])


def _phase_c(lt_ref, lb_ref, sums_ref, csc_ref, ctc_ref, p_ref, x_ref,
             out_ref, pn2_scr):
    i = pl.program_id(0)

    @pl.when(i == 0)
    def _():
        momentum = 0.999
        p0 = p_ref[...]
        sums = sums_ref[...]
        counts = ctc_ref[...] + lax.dot_general(
            csc_ref[...], jnp.ones((_NW, 1), jnp.float32),
            (((0,), (0,)), ((), ())),
            preferred_element_type=jnp.float32)
        centroids = jnp.where(counts > 0.0,
                              sums / jnp.maximum(counts, 1.0), p0)
        new_p = centroids / jnp.maximum(
            jnp.sqrt(jnp.sum(centroids * centroids, axis=1, keepdims=True)),
            1e-12)
        p_upd = momentum * p0 + (1.0 - momentum) * new_p
        pn2_scr[...] = p_upd / jnp.maximum(
            jnp.sqrt(jnp.sum(p_upd * p_upd, axis=1, keepdims=True)), 1e-8)

    x = x_ref[...]
    pn2 = pn2_scr[...]
    rn = jnp.sqrt(jnp.sum(x * x, axis=1, keepdims=True))
    xn = x / jnp.maximum(rn, 1e-8)
    sims2 = lax.dot_general(xn.astype(jnp.bfloat16), pn2.astype(jnp.bfloat16),
                            (((1,), (1,)), ((), ())),
                            preferred_element_type=jnp.float32)
    mx = jnp.max(sims2, axis=1, keepdims=True)
    dists = jnp.clip(1.0 - mx, 0.0, 2.0)
    tau = jnp.exp(lt_ref[0])
    alpha = jax.nn.sigmoid(lb_ref[0])
    novelty = 1.0 - jnp.exp(-tau * dists)
    scale = jnp.clip(1.0 - alpha + alpha * novelty, 0.1, 10.0)
    y = x * scale
    out_ref[...] = 0.5 * y * (
        1.0 + jnp.tanh(_SQRT_2_OVER_PI * (y + 0.044715 * y * y * y)))


def kernel(x, P, log_tau, log_blend):
    B, T, D = x.shape
    K = P.shape[0]
    N = B * T
    xf = x.reshape(N, D)
    n_tiles = N // _TT

    assign, sums, counts_tc = pl.pallas_call(
        _phase_a,
        grid=(n_tiles,),
        in_specs=[
            pl.BlockSpec((_TT, D), lambda i: (i, 0)),
            pl.BlockSpec((K, D), lambda i: (0, 0)),
        ],
        out_specs=[
            pl.BlockSpec((1, 1, _TT), lambda i: (i, 0, 0)),
            pl.BlockSpec((K, D), lambda i: (0, 0)),
            pl.BlockSpec((K, 1), lambda i: (0, 0)),
        ],
        out_shape=[
            jax.ShapeDtypeStruct((n_tiles, 1, _TT), jnp.int32),
            jax.ShapeDtypeStruct((K, D), jnp.float32),
            jax.ShapeDtypeStruct((K, 1), jnp.float32),
        ],
    )(xf, P)

    counts_sc = pl.kernel(
        _counts_body,
        out_type=jax.ShapeDtypeStruct((_NW, _K), jnp.float32),
        mesh=plsc.VectorSubcoreMesh(core_axis_name="c", subcore_axis_name="s"),
        compiler_params=pltpu.CompilerParams(needs_layout_passes=False),
        scratch_types=[
            pltpu.VMEM((_SC_TOK,), jnp.int32),
            pltpu.VMEM((_LANES, _K), jnp.float32),
            pltpu.VMEM((_K,), jnp.float32),
        ],
    )(assign.reshape(N)[:_HALF])

    lt = jnp.reshape(log_tau, (1,))
    lb = jnp.reshape(log_blend, (1,))
    out = pl.pallas_call(
        _phase_c,
        grid=(n_tiles,),
        in_specs=[
            pl.BlockSpec(memory_space=pltpu.SMEM),
            pl.BlockSpec(memory_space=pltpu.SMEM),
            pl.BlockSpec((K, D), lambda i: (0, 0)),
            pl.BlockSpec((_NW, K), lambda i: (0, 0)),
            pl.BlockSpec((K, 1), lambda i: (0, 0)),
            pl.BlockSpec((K, D), lambda i: (0, 0)),
            pl.BlockSpec((_TT, D), lambda i: (i, 0)),
        ],
        out_specs=pl.BlockSpec((_TT, D), lambda i: (i, 0)),
        out_shape=jax.ShapeDtypeStruct((N, D), jnp.float32),
        scratch_shapes=[pltpu.VMEM((K, D), jnp.float32)],
    )(lt, lb, sums, counts_sc, counts_tc, P, xf)

    return out.reshape(B, T, D)


# speedup vs baseline: 1.3405x; 1.0181x over previous
import math

import jax
import jax.numpy as jnp
from jax import lax
from jax.experimental import pallas as pl
from jax.experimental.pallas import tpu as pltpu
from jax.experimental.pallas import tpu_sc as plsc

_SQRT_2_OVER_PI = math.sqrt(2.0 / math.pi)

_N = 8192
_D = 768
_K = 512
_NC = 2
_NS = 16
_NW = _NC * _NS
_LANES = 16
_HALF = _N // 2
_SC_W = _NS
_SC_TOK = _HALF // _SC_W
_TT = 1024
_HALF_TILES = _HALF // _TT


def _phase_a(x_ref, p_ref, assign_ref, sums_ref, counts_ref):
    i = pl.program_id(0)
    x = x_ref[...]
    p0 = p_ref[...]
    pn = p0 / jnp.maximum(
        jnp.sqrt(jnp.sum(p0 * p0, axis=1, keepdims=True)), 1e-12)
    rn = jnp.sqrt(jnp.sum(x * x, axis=1, keepdims=True))
    xn = x / jnp.maximum(rn, 1e-8)
    sims = lax.dot_general(xn.astype(jnp.bfloat16), pn.astype(jnp.bfloat16),
                           (((1,), (1,)), ((), ())),
                           preferred_element_type=jnp.float32)
    m = jnp.max(sims, axis=1, keepdims=True)
    k_iota = lax.broadcasted_iota(jnp.int32, sims.shape, 1)
    idx = jnp.min(jnp.where(sims >= m, k_iota, sims.shape[1]), axis=1)
    assign_ref[...] = idx.reshape(assign_ref.shape)
    e = (k_iota == idx[:, None]).astype(jnp.bfloat16)
    part = lax.dot_general(e, x.astype(jnp.bfloat16), (((0,), (0,)), ((), ())),
                           preferred_element_type=jnp.float32)
    cpart = lax.dot_general(e, jnp.ones((x.shape[0], 1), jnp.bfloat16),
                            (((0,), (0,)), ((), ())),
                            preferred_element_type=jnp.float32)

    @pl.when(i == 0)
    def _():
        sums_ref[...] = jnp.zeros_like(sums_ref)
        counts_ref[...] = jnp.zeros_like(counts_ref)

    sums_ref[...] += part

    @pl.when(i >= _HALF_TILES)
    def _():
        counts_ref[...] += cpart


def _counts_body(a_hbm, counts_hbm, idx_v, hist_v, out_v):
    s = lax.axis_index("s")
    w = s
    pltpu.sync_copy(a_hbm.at[pl.ds(w * _SC_TOK, _SC_TOK)], idx_v)

    def zero(i, _):
        def zr(r, _):
            hist_v[r, pl.ds(i * _LANES, _LANES)] = jnp.zeros(
                (_LANES,), jnp.float32)
            return 0
        lax.fori_loop(0, _LANES, zr, 0)
        return 0
    lax.fori_loop(0, _K // _LANES, zero, 0)

    lane_iota = lax.iota(jnp.int32, _LANES)
    ones16 = jnp.ones((_LANES,), jnp.float32)

    def accum(i, _):
        iv = idx_v[pl.ds(i * _LANES, _LANES)]
        plsc.addupdate_scatter(hist_v, [lane_iota, iv], ones16)
        return 0
    lax.fori_loop(0, _SC_TOK // _LANES, accum, 0)

    def reduce_cols(j, _):
        def rr(r, acc):
            return acc + hist_v[r, pl.ds(j * _LANES, _LANES)]
        out_v[pl.ds(j * _LANES, _LANES)] = lax.fori_loop(
            0, _LANES, rr, jnp.zeros((_LANES,), jnp.float32))
        return 0
    lax.fori_loop(0, _K // _LANES, reduce_cols, 0)
    pltpu.sync_copy(out_v, counts_hbm.at[w])


def _phase_c(lt_ref, lb_ref, sums_ref, csc_ref, ctc_ref, p_ref, x_ref,
             out_ref, pn2_scr):
    i = pl.program_id(0)

    @pl.when(i == 0)
    def _():
        momentum = 0.999
        p0 = p_ref[...]
        sums = sums_ref[...]
        counts = ctc_ref[...] + lax.dot_general(
            csc_ref[...], jnp.ones((_SC_W, 1), jnp.float32),
            (((0,), (0,)), ((), ())),
            preferred_element_type=jnp.float32)
        centroids = jnp.where(counts > 0.0,
                              sums / jnp.maximum(counts, 1.0), p0)
        new_p = centroids / jnp.maximum(
            jnp.sqrt(jnp.sum(centroids * centroids, axis=1, keepdims=True)),
            1e-12)
        p_upd = momentum * p0 + (1.0 - momentum) * new_p
        pn2_scr[...] = p_upd / jnp.maximum(
            jnp.sqrt(jnp.sum(p_upd * p_upd, axis=1, keepdims=True)), 1e-8)

    x = x_ref[...]
    pn2 = pn2_scr[...]
    rn = jnp.sqrt(jnp.sum(x * x, axis=1, keepdims=True))
    xn = x / jnp.maximum(rn, 1e-8)
    sims2 = lax.dot_general(xn.astype(jnp.bfloat16), pn2.astype(jnp.bfloat16),
                            (((1,), (1,)), ((), ())),
                            preferred_element_type=jnp.float32)
    mx = jnp.max(sims2, axis=1, keepdims=True)
    dists = jnp.clip(1.0 - mx, 0.0, 2.0)
    tau = jnp.exp(lt_ref[0])
    alpha = jax.nn.sigmoid(lb_ref[0])
    novelty = 1.0 - jnp.exp(-tau * dists)
    scale = jnp.clip(1.0 - alpha + alpha * novelty, 0.1, 10.0)
    y = x * scale
    out_ref[...] = 0.5 * y * (
        1.0 + jnp.tanh(_SQRT_2_OVER_PI * (y + 0.044715 * y * y * y)))


def kernel(x, P, log_tau, log_blend):
    B, T, D = x.shape
    K = P.shape[0]
    N = B * T
    xf = x.reshape(N, D)
    n_tiles = N // _TT

    assign, sums, counts_tc = pl.pallas_call(
        _phase_a,
        grid=(n_tiles,),
        in_specs=[
            pl.BlockSpec((_TT, D), lambda i: (i, 0)),
            pl.BlockSpec((K, D), lambda i: (0, 0)),
        ],
        out_specs=[
            pl.BlockSpec((1, 1, _TT), lambda i: (i, 0, 0)),
            pl.BlockSpec((K, D), lambda i: (0, 0)),
            pl.BlockSpec((K, 1), lambda i: (0, 0)),
        ],
        out_shape=[
            jax.ShapeDtypeStruct((n_tiles, 1, _TT), jnp.int32),
            jax.ShapeDtypeStruct((K, D), jnp.float32),
            jax.ShapeDtypeStruct((K, 1), jnp.float32),
        ],
    )(xf, P)

    counts_sc = pl.kernel(
        _counts_body,
        out_type=jax.ShapeDtypeStruct((_SC_W, _K), jnp.float32),
        mesh=plsc.VectorSubcoreMesh(core_axis_name="c", subcore_axis_name="s",
                                    num_cores=1),
        compiler_params=pltpu.CompilerParams(needs_layout_passes=False),
        scratch_types=[
            pltpu.VMEM((_SC_TOK,), jnp.int32),
            pltpu.VMEM((_LANES, _K), jnp.float32),
            pltpu.VMEM((_K,), jnp.float32),
        ],
    )(assign.reshape(N)[:_HALF])

    lt = jnp.reshape(log_tau, (1,))
    lb = jnp.reshape(log_blend, (1,))
    out = pl.pallas_call(
        _phase_c,
        grid=(n_tiles,),
        in_specs=[
            pl.BlockSpec(memory_space=pltpu.SMEM),
            pl.BlockSpec(memory_space=pltpu.SMEM),
            pl.BlockSpec((K, D), lambda i: (0, 0)),
            pl.BlockSpec((_SC_W, K), lambda i: (0, 0)),
            pl.BlockSpec((K, 1), lambda i: (0, 0)),
            pl.BlockSpec((K, D), lambda i: (0, 0)),
            pl.BlockSpec((_TT, D), lambda i: (i, 0)),
        ],
        out_specs=pl.BlockSpec((_TT, D), lambda i: (i, 0)),
        out_shape=jax.ShapeDtypeStruct((N, D), jnp.float32),
        scratch_shapes=[pltpu.VMEM((K, D), jnp.float32)],
    )(lt, lb, sums, counts_sc, counts_tc, P, xf)

    return out.reshape(B, T, D)


# speedup vs baseline: 1.3621x; 1.0161x over previous
import math

import jax
import jax.numpy as jnp
from jax import lax
from jax.experimental import pallas as pl
from jax.experimental.pallas import tpu as pltpu
from jax.experimental.pallas import tpu_sc as plsc

_SQRT_2_OVER_PI = math.sqrt(2.0 / math.pi)

_N = 8192
_D = 768
_K = 512
_NC = 2
_NS = 16
_NW = _NC * _NS
_LANES = 16
_HALF = _N // 2
_SC_W = _NS
_SC_TOK = _HALF // _SC_W
_TT = 1024
_HALF_TILES = _HALF // _TT


def _phase_a(x_ref, p_ref, assign_ref, sums_ref, counts_ref):
    i = pl.program_id(0)
    x = x_ref[...]
    p0 = p_ref[...]
    pn = p0 / jnp.maximum(
        jnp.sqrt(jnp.sum(p0 * p0, axis=1, keepdims=True)), 1e-12)
    rn = jnp.sqrt(jnp.sum(x * x, axis=1, keepdims=True))
    xn = x / jnp.maximum(rn, 1e-8)
    sims = lax.dot_general(xn.astype(jnp.bfloat16), pn.astype(jnp.bfloat16),
                           (((1,), (1,)), ((), ())),
                           preferred_element_type=jnp.float32)
    m = jnp.max(sims, axis=1, keepdims=True)
    k_iota = lax.broadcasted_iota(jnp.int32, sims.shape, 1)
    idx = jnp.min(jnp.where(sims >= m, k_iota, sims.shape[1]), axis=1)
    assign_ref[...] = idx.reshape(assign_ref.shape)
    e = (k_iota == idx[:, None]).astype(jnp.bfloat16)
    part = lax.dot_general(e, x.astype(jnp.bfloat16), (((0,), (0,)), ((), ())),
                           preferred_element_type=jnp.float32)
    cpart = lax.dot_general(e, jnp.ones((x.shape[0], 1), jnp.bfloat16),
                            (((0,), (0,)), ((), ())),
                            preferred_element_type=jnp.float32)

    @pl.when(i == 0)
    def _():
        sums_ref[...] = jnp.zeros_like(sums_ref)
        counts_ref[...] = jnp.zeros_like(counts_ref)

    sums_ref[...] += part

    @pl.when(i >= _HALF_TILES)
    def _():
        counts_ref[...] += cpart


def _counts_body(a_hbm, counts_hbm, idx_v, hist_v):
    w = lax.axis_index("s")
    pltpu.sync_copy(a_hbm.at[pl.ds(w * _SC_TOK, _SC_TOK)], idx_v)

    z16 = jnp.zeros((_LANES,), jnp.float32)
    for r in range(_LANES):
        for j in range(_K // _LANES):
            hist_v[r, pl.ds(j * _LANES, _LANES)] = z16

    lane_iota = lax.iota(jnp.int32, _LANES)
    ones16 = jnp.ones((_LANES,), jnp.float32)
    for i in range(_SC_TOK // _LANES):
        iv = idx_v[pl.ds(i * _LANES, _LANES)]
        plsc.addupdate_scatter(hist_v, [lane_iota, iv], ones16)
    pltpu.sync_copy(hist_v, counts_hbm.at[w])


def _phase_c(lt_ref, lb_ref, sums_ref, csc_ref, ctc_ref, p_ref, x_ref,
             out_ref, pn2_scr):
    i = pl.program_id(0)

    @pl.when(i == 0)
    def _():
        momentum = 0.999
        p0 = p_ref[...]
        sums = sums_ref[...]
        counts = ctc_ref[...] + lax.dot_general(
            csc_ref[...], jnp.ones((_SC_W * _LANES, 1), jnp.float32),
            (((0,), (0,)), ((), ())),
            preferred_element_type=jnp.float32)
        centroids = jnp.where(counts > 0.0,
                              sums / jnp.maximum(counts, 1.0), p0)
        new_p = centroids / jnp.maximum(
            jnp.sqrt(jnp.sum(centroids * centroids, axis=1, keepdims=True)),
            1e-12)
        p_upd = momentum * p0 + (1.0 - momentum) * new_p
        pn2_scr[...] = p_upd / jnp.maximum(
            jnp.sqrt(jnp.sum(p_upd * p_upd, axis=1, keepdims=True)), 1e-8)

    x = x_ref[...]
    pn2 = pn2_scr[...]
    rn = jnp.sqrt(jnp.sum(x * x, axis=1, keepdims=True))
    xn = x / jnp.maximum(rn, 1e-8)
    sims2 = lax.dot_general(xn.astype(jnp.bfloat16), pn2.astype(jnp.bfloat16),
                            (((1,), (1,)), ((), ())),
                            preferred_element_type=jnp.float32)
    mx = jnp.max(sims2, axis=1, keepdims=True)
    dists = jnp.clip(1.0 - mx, 0.0, 2.0)
    tau = jnp.exp(lt_ref[0])
    alpha = jax.nn.sigmoid(lb_ref[0])
    novelty = 1.0 - jnp.exp(-tau * dists)
    scale = jnp.clip(1.0 - alpha + alpha * novelty, 0.1, 10.0)
    y = x * scale
    out_ref[...] = 0.5 * y * (
        1.0 + jnp.tanh(_SQRT_2_OVER_PI * (y + 0.044715 * y * y * y)))


def kernel(x, P, log_tau, log_blend):
    B, T, D = x.shape
    K = P.shape[0]
    N = B * T
    xf = x.reshape(N, D)
    n_tiles = N // _TT

    assign, sums, counts_tc = pl.pallas_call(
        _phase_a,
        grid=(n_tiles,),
        in_specs=[
            pl.BlockSpec((_TT, D), lambda i: (i, 0)),
            pl.BlockSpec((K, D), lambda i: (0, 0)),
        ],
        out_specs=[
            pl.BlockSpec((1, 1, _TT), lambda i: (i, 0, 0)),
            pl.BlockSpec((K, D), lambda i: (0, 0)),
            pl.BlockSpec((K, 1), lambda i: (0, 0)),
        ],
        out_shape=[
            jax.ShapeDtypeStruct((n_tiles, 1, _TT), jnp.int32),
            jax.ShapeDtypeStruct((K, D), jnp.float32),
            jax.ShapeDtypeStruct((K, 1), jnp.float32),
        ],
    )(xf, P)

    counts_sc = pl.kernel(
        _counts_body,
        out_type=jax.ShapeDtypeStruct((_SC_W, _LANES, _K), jnp.float32),
        mesh=plsc.VectorSubcoreMesh(core_axis_name="c", subcore_axis_name="s",
                                    num_cores=1),
        compiler_params=pltpu.CompilerParams(needs_layout_passes=False),
        scratch_types=[
            pltpu.VMEM((_SC_TOK,), jnp.int32),
            pltpu.VMEM((_LANES, _K), jnp.float32),
        ],
    )(assign.reshape(N)[:_HALF])
    counts_sc = counts_sc.reshape(_SC_W * _LANES, _K)

    lt = jnp.reshape(log_tau, (1,))
    lb = jnp.reshape(log_blend, (1,))
    out = pl.pallas_call(
        _phase_c,
        grid=(n_tiles,),
        in_specs=[
            pl.BlockSpec(memory_space=pltpu.SMEM),
            pl.BlockSpec(memory_space=pltpu.SMEM),
            pl.BlockSpec((K, D), lambda i: (0, 0)),
            pl.BlockSpec((_SC_W * _LANES, K), lambda i: (0, 0)),
            pl.BlockSpec((K, 1), lambda i: (0, 0)),
            pl.BlockSpec((K, D), lambda i: (0, 0)),
            pl.BlockSpec((_TT, D), lambda i: (i, 0)),
        ],
        out_specs=pl.BlockSpec((_TT, D), lambda i: (i, 0)),
        out_shape=jax.ShapeDtypeStruct((N, D), jnp.float32),
        scratch_shapes=[pltpu.VMEM((K, D), jnp.float32)],
    )(lt, lb, sums, counts_sc, counts_tc, P, xf)

    return out.reshape(B, T, D)


# speedup vs baseline: 1.3800x; 1.0132x over previous
import math

import jax
import jax.numpy as jnp
from jax import lax
from jax.experimental import pallas as pl
from jax.experimental.pallas import tpu as pltpu
from jax.experimental.pallas import tpu_sc as plsc

_SQRT_2_OVER_PI = math.sqrt(2.0 / math.pi)

_N = 8192
_D = 768
_K = 512
_NC = 2
_NS = 16
_NW = _NC * _NS
_LANES = 16
_HALF = _N // 2
_SC_W = _NS
_SC_TOK = _HALF // _SC_W
_TT = 1024
_HALF_TILES = _HALF // _TT


def _phase_a(x_ref, p_ref, assign_ref, sums_ref, counts_ref, pn_scr):
    i = pl.program_id(0)

    @pl.when(i == 0)
    def _():
        p0 = p_ref[...]
        pn_scr[...] = (p0 / jnp.maximum(
            jnp.sqrt(jnp.sum(p0 * p0, axis=1, keepdims=True)),
            1e-12)).astype(jnp.bfloat16)

    x = x_ref[...]
    rn = jnp.sqrt(jnp.sum(x * x, axis=1, keepdims=True))
    xn = x / jnp.maximum(rn, 1e-8)
    sims = lax.dot_general(xn.astype(jnp.bfloat16), pn_scr[...],
                           (((1,), (1,)), ((), ())),
                           preferred_element_type=jnp.float32)
    m = jnp.max(sims, axis=1, keepdims=True)
    k_iota = lax.broadcasted_iota(jnp.int32, sims.shape, 1)
    idx = jnp.min(jnp.where(sims >= m, k_iota, sims.shape[1]), axis=1)
    assign_ref[...] = idx.reshape(assign_ref.shape)
    e = (k_iota == idx[:, None]).astype(jnp.bfloat16)
    part = lax.dot_general(e, x.astype(jnp.bfloat16), (((0,), (0,)), ((), ())),
                           preferred_element_type=jnp.float32)
    cpart = lax.dot_general(e, jnp.ones((x.shape[0], 1), jnp.bfloat16),
                            (((0,), (0,)), ((), ())),
                            preferred_element_type=jnp.float32)

    @pl.when(i == 0)
    def _():
        sums_ref[...] = jnp.zeros_like(sums_ref)
        counts_ref[...] = jnp.zeros_like(counts_ref)

    sums_ref[...] += part

    @pl.when(i >= _HALF_TILES)
    def _():
        counts_ref[...] += cpart


def _counts_body(a_hbm, counts_hbm, idx_v, hist_v):
    w = lax.axis_index("s")
    pltpu.sync_copy(a_hbm.at[pl.ds(w * _SC_TOK, _SC_TOK)], idx_v)

    z16 = jnp.zeros((_LANES,), jnp.float32)
    for r in range(_LANES):
        for j in range(_K // _LANES):
            hist_v[r, pl.ds(j * _LANES, _LANES)] = z16

    lane_iota = lax.iota(jnp.int32, _LANES)
    ones16 = jnp.ones((_LANES,), jnp.float32)
    for i in range(_SC_TOK // _LANES):
        iv = idx_v[pl.ds(i * _LANES, _LANES)]
        plsc.addupdate_scatter(hist_v, [lane_iota, iv], ones16)
    pltpu.sync_copy(hist_v, counts_hbm.at[w])


def _phase_c(lt_ref, lb_ref, sums_ref, csc_ref, ctc_ref, p_ref, x_ref,
             out_ref, pn2_scr):
    i = pl.program_id(0)

    @pl.when(i == 0)
    def _():
        momentum = 0.999
        p0 = p_ref[...]
        sums = sums_ref[...]
        counts = ctc_ref[...] + lax.dot_general(
            csc_ref[...], jnp.ones((_SC_W * _LANES, 1), jnp.float32),
            (((0,), (0,)), ((), ())),
            preferred_element_type=jnp.float32)
        centroids = jnp.where(counts > 0.0,
                              sums / jnp.maximum(counts, 1.0), p0)
        new_p = centroids / jnp.maximum(
            jnp.sqrt(jnp.sum(centroids * centroids, axis=1, keepdims=True)),
            1e-12)
        p_upd = momentum * p0 + (1.0 - momentum) * new_p
        pn2_scr[...] = (p_upd / jnp.maximum(
            jnp.sqrt(jnp.sum(p_upd * p_upd, axis=1, keepdims=True)),
            1e-8)).astype(jnp.bfloat16)

    x = x_ref[...]
    rn = jnp.sqrt(jnp.sum(x * x, axis=1, keepdims=True))
    xn = x / jnp.maximum(rn, 1e-8)
    sims2 = lax.dot_general(xn.astype(jnp.bfloat16), pn2_scr[...],
                            (((1,), (1,)), ((), ())),
                            preferred_element_type=jnp.float32)
    mx = jnp.max(sims2, axis=1, keepdims=True)
    dists = jnp.clip(1.0 - mx, 0.0, 2.0)
    tau = jnp.exp(lt_ref[0])
    alpha = jax.nn.sigmoid(lb_ref[0])
    novelty = 1.0 - jnp.exp(-tau * dists)
    scale = jnp.clip(1.0 - alpha + alpha * novelty, 0.1, 10.0)
    y = x * scale
    out_ref[...] = 0.5 * y * (
        1.0 + jnp.tanh(_SQRT_2_OVER_PI * (y + 0.044715 * y * y * y)))


def kernel(x, P, log_tau, log_blend):
    B, T, D = x.shape
    K = P.shape[0]
    N = B * T
    xf = x.reshape(N, D)
    n_tiles = N // _TT

    assign, sums, counts_tc = pl.pallas_call(
        _phase_a,
        grid=(n_tiles,),
        in_specs=[
            pl.BlockSpec((_TT, D), lambda i: (i, 0)),
            pl.BlockSpec((K, D), lambda i: (0, 0)),
        ],
        out_specs=[
            pl.BlockSpec((1, 1, _TT), lambda i: (i, 0, 0)),
            pl.BlockSpec((K, D), lambda i: (0, 0)),
            pl.BlockSpec((K, 1), lambda i: (0, 0)),
        ],
        out_shape=[
            jax.ShapeDtypeStruct((n_tiles, 1, _TT), jnp.int32),
            jax.ShapeDtypeStruct((K, D), jnp.float32),
            jax.ShapeDtypeStruct((K, 1), jnp.float32),
        ],
        scratch_shapes=[pltpu.VMEM((K, D), jnp.bfloat16)],
    )(xf, P)

    counts_sc = pl.kernel(
        _counts_body,
        out_type=jax.ShapeDtypeStruct((_SC_W, _LANES, _K), jnp.float32),
        mesh=plsc.VectorSubcoreMesh(core_axis_name="c", subcore_axis_name="s",
                                    num_cores=1),
        compiler_params=pltpu.CompilerParams(needs_layout_passes=False),
        scratch_types=[
            pltpu.VMEM((_SC_TOK,), jnp.int32),
            pltpu.VMEM((_LANES, _K), jnp.float32),
        ],
    )(assign.reshape(N)[:_HALF])
    counts_sc = counts_sc.reshape(_SC_W * _LANES, _K)

    lt = jnp.reshape(log_tau, (1,))
    lb = jnp.reshape(log_blend, (1,))
    out = pl.pallas_call(
        _phase_c,
        grid=(n_tiles,),
        in_specs=[
            pl.BlockSpec(memory_space=pltpu.SMEM),
            pl.BlockSpec(memory_space=pltpu.SMEM),
            pl.BlockSpec((K, D), lambda i: (0, 0)),
            pl.BlockSpec((_SC_W * _LANES, K), lambda i: (0, 0)),
            pl.BlockSpec((K, 1), lambda i: (0, 0)),
            pl.BlockSpec((K, D), lambda i: (0, 0)),
            pl.BlockSpec((_TT, D), lambda i: (i, 0)),
        ],
        out_specs=pl.BlockSpec((_TT, D), lambda i: (i, 0)),
        out_shape=jax.ShapeDtypeStruct((N, D), jnp.float32),
        scratch_shapes=[pltpu.VMEM((K, D), jnp.bfloat16)],
    )(lt, lb, sums, counts_sc, counts_tc, P, xf)

    return out.reshape(B, T, D)
